# baseline (speedup 1.0000x reference)
import numpy as np
import jax
import jax.numpy as jnp
from jax.experimental import pallas as pl
from jax.experimental.pallas import tpu as pltpu

N = 10000
F = 128
C = 6
BI = 400
NB = N // BI
WCH = 1664
NCH = N // WCH
BAND_OFF = NCH * WCH
BAND_W = N - BAND_OFF
BI2 = 2000
NB2 = N // BI2
SEG = BI2 // BI


def _pass1_kernel(adj_ref, zeff_ref, W1_ref, b1_ref, W2_ref,
                  m_ref, part_ref, band_ref, mscr_ref):
    i = pl.program_id(0)
    slab = adj_ref[...]
    rowids = jax.lax.broadcasted_iota(jnp.int32, (N, 1), 0)
    keep = rowids < ((i * BI // BI2) * BI2 // WCH) * WCH
    mm = jnp.where(keep, mscr_ref[...], 0.0)
    ones = jnp.ones((N, 1), jnp.float32)
    mm7 = jnp.concatenate([mm, ones], axis=1)
    acc = jnp.dot(slab, mm7,
                  preferred_element_type=jnp.float32)
    s = acc[:, C:C + 1]
    r = jnp.dot(zeff_ref[...], W1_ref[...],
                preferred_element_type=jnp.float32)
    h = jax.nn.relu(s * r + b1_ref[...])
    m_i = jnp.dot(h, W2_ref[...],
                  preferred_element_type=jnp.float32)
    mscr_ref[pl.ds(i * BI, BI), :] = m_i
    m_ref[...] = m_i
    part_ref[...] = acc[:, :C]
    band_ref[...] = slab[:, BAND_OFF:N]


def _pass2_kernel(iR, cR, fR, lR,
                  adj_ref, m_ref, part_ref, band_ref, mband_ref, b2_ref,
                  out_ref, acc_ref):
    t = pl.program_id(0)

    @pl.when(fR[t] == 1)
    def _():
        acc_ref[...] = jnp.zeros((BI2, C), jnp.float32)

    acc_ref[...] += jnp.dot(adj_ref[...], m_ref[...],
                            preferred_element_type=jnp.float32)

    @pl.when(lR[t] == 1)
    def _():
        band = jnp.dot(band_ref[...], mband_ref[...],
                       preferred_element_type=jnp.float32)
        o = acc_ref[...] + band + part_ref[...] + b2_ref[...]
        mx = jnp.max(o, axis=1, keepdims=True)
        lse = jnp.log(jnp.sum(jnp.exp(o - mx), axis=1, keepdims=True)) + mx
        out_ref[...] = o - lse


def _pass2_schedule():
    is_, cs, fs, ls = [], [], [], []
    for i in range(NB2):
        c0 = (i * BI2) // WCH
        for c in range(c0, NCH):
            is_.append(i)
            cs.append(c)
            fs.append(1 if c == c0 else 0)
            ls.append(1 if c == NCH - 1 else 0)
    mk = lambda v: jnp.asarray(np.array(v, dtype=np.int32))
    return mk(is_), mk(cs), mk(fs), mk(ls), len(is_)


_I_ARR, _C_ARR, _F_ARR, _L_ARR, _T2 = _pass2_schedule()


@jax.jit
def kernel(adj, z, W1, b1, W2, b2, n_nodes):
    zero_residual = (jnp.asarray(n_nodes) - N).astype(jnp.float32)
    z_eff = z + zero_residual
    b1r = b1.reshape(1, F)
    b2r = b2.reshape(1, C)

    M, partial, band = pl.pallas_call(
        _pass1_kernel,
        grid=(NB,),
        in_specs=[
            pl.BlockSpec((BI, N), lambda i: (i, 0)),
            pl.BlockSpec((1, F), lambda i: (0, 0)),
            pl.BlockSpec((F, F), lambda i: (0, 0)),
            pl.BlockSpec((1, F), lambda i: (0, 0)),
            pl.BlockSpec((F, C), lambda i: (0, 0)),
        ],
        out_specs=[
            pl.BlockSpec((BI, C), lambda i: (i, 0)),
            pl.BlockSpec((BI, C), lambda i: (i, 0)),
            pl.BlockSpec((BI, BAND_W), lambda i: (i, 0)),
        ],
        out_shape=[
            jax.ShapeDtypeStruct((N, C), jnp.float32),
            jax.ShapeDtypeStruct((N, C), jnp.float32),
            jax.ShapeDtypeStruct((N, BAND_W), jnp.float32),
        ],
        scratch_shapes=[pltpu.VMEM((N, C), jnp.float32)],
    )(adj, z_eff, W1, b1r, W2)

    m_band = jax.lax.slice(M, (BAND_OFF, 0), (N, C))
    grid_spec = pltpu.PrefetchScalarGridSpec(
        num_scalar_prefetch=4,
        grid=(_T2,),
        in_specs=[
            pl.BlockSpec((BI2, WCH), lambda t, iR, cR, *_: (iR[t], cR[t])),
            pl.BlockSpec((WCH, C), lambda t, iR, cR, *_: (cR[t], 0)),
            pl.BlockSpec((BI2, C), lambda t, iR, cR, *_: (iR[t], 0)),
            pl.BlockSpec((BI2, BAND_W), lambda t, iR, *_: (iR[t], 0)),
            pl.BlockSpec((BAND_W, C), lambda t, *_: (0, 0)),
            pl.BlockSpec((1, C), lambda t, *_: (0, 0)),
        ],
        out_specs=pl.BlockSpec((BI2, C), lambda t, iR, *_: (iR[t], 0)),
        scratch_shapes=[pltpu.VMEM((BI2, C), jnp.float32)],
    )
    out = pl.pallas_call(
        _pass2_kernel,
        grid_spec=grid_spec,
        out_shape=jax.ShapeDtypeStruct((N, C), jnp.float32),
    )(_I_ARR, _C_ARR, _F_ARR, _L_ARR, adj, M, partial, band, m_band, b2r)
    return out


# speedup vs baseline: 1.0271x; 1.0271x over previous
import numpy as np
import jax
import jax.numpy as jnp
from jax.experimental import pallas as pl
from jax.experimental.pallas import tpu as pltpu

N = 10000
F = 128
C = 6
BI = 400
NB = N // BI
WCH = 768
NCH = N // WCH
BAND_OFF = NCH * WCH
BAND_W = N - BAND_OFF
BI2 = 2000
NB2 = N // BI2
MCOL, PCOL, BCOL = 0, 6, 16


def _fused_kernel(pR, sR, gR, cR, fR, lR,
                  slab_ref, chunk_ref, zeff_ref, W1_ref, b1_ref, W2_ref,
                  b2_ref, out_ref, combo_ref, acc_ref):
    t = pl.program_id(0)

    @pl.when(pR[t] == 0)
    def _phase1():
        i = sR[t]
        slab = slab_ref[...]
        rowids = jax.lax.broadcasted_iota(jnp.int32, (N, 1), 0)
        keep = rowids < ((i * BI // BI2) * BI2 // WCH) * WCH
        mm = jnp.where(keep, combo_ref[:, MCOL:MCOL + C], 0.0)
        ones = jnp.ones((N, 1), jnp.float32)
        mm7 = jnp.concatenate([mm, ones], axis=1)
        acc7 = jnp.dot(slab, mm7,
                       preferred_element_type=jnp.float32)
        s = acc7[:, C:C + 1]
        r = jnp.dot(zeff_ref[...], W1_ref[...],
                    preferred_element_type=jnp.float32)
        h = jax.nn.relu(s * r + b1_ref[...])
        m_i = jnp.dot(h, W2_ref[...],
                      preferred_element_type=jnp.float32)
        combo_ref[pl.ds(i * BI, BI), MCOL:MCOL + C] = m_i
        combo_ref[pl.ds(i * BI, BI), PCOL:PCOL + C] = acc7[:, :C]
        combo_ref[pl.ds(i * BI, BI), BCOL:BCOL + BAND_W] = slab[:, BAND_OFF:N]

    @pl.when(pR[t] == 1)
    def _phase2():
        seg = gR[t]
        c = cR[t]

        @pl.when(fR[t] == 1)
        def _():
            acc_ref[...] = jnp.zeros((BI2, C), jnp.float32)

        mslice = combo_ref[pl.ds(c * WCH, WCH), MCOL:MCOL + C]
        acc_ref[...] += jnp.dot(chunk_ref[...], mslice,
                                preferred_element_type=jnp.float32)

        @pl.when(lR[t] == 1)
        def _():
            band = combo_ref[pl.ds(seg * BI2, BI2), BCOL:BCOL + BAND_W]
            mband = combo_ref[BAND_OFF:N, MCOL:MCOL + C]
            part = combo_ref[pl.ds(seg * BI2, BI2), PCOL:PCOL + C]
            o = (acc_ref[...] + part + b2_ref[...]
                 + jnp.dot(band, mband, preferred_element_type=jnp.float32))
            mx = jnp.max(o, axis=1, keepdims=True)
            lse = jnp.log(jnp.sum(jnp.exp(o - mx), axis=1,
                                  keepdims=True)) + mx
            out_ref[...] = o - lse


def _schedule():
    ps, ss, gs, cs, fs, ls = [], [], [], [], [], []
    for i in range(NB):
        ps.append(0)
        ss.append(i)
        gs.append(0)
        cs.append(0)
        fs.append(0)
        ls.append(0)
    for g in range(NB2):
        c0 = (g * BI2) // WCH
        for c in range(c0, NCH):
            ps.append(1)
            ss.append(NB - 1)
            gs.append(g)
            cs.append(c)
            fs.append(1 if c == c0 else 0)
            ls.append(1 if c == NCH - 1 else 0)
    mk = lambda v: jnp.asarray(np.array(v, dtype=np.int32))
    return (mk(ps), mk(ss), mk(gs), mk(cs), mk(fs), mk(ls), len(ps))


_P_ARR, _S_ARR, _G_ARR, _C_ARR, _F_ARR, _L_ARR, _T = _schedule()


@jax.jit
def kernel(adj, z, W1, b1, W2, b2, n_nodes):
    zero_residual = (jnp.asarray(n_nodes) - N).astype(jnp.float32)
    z_eff = z + zero_residual
    b1r = b1.reshape(1, F)
    b2r = b2.reshape(1, C)

    grid_spec = pltpu.PrefetchScalarGridSpec(
        num_scalar_prefetch=6,
        grid=(_T,),
        in_specs=[
            pl.BlockSpec((BI, N), lambda t, pR, sR, *_: (sR[t], 0)),
            pl.BlockSpec((BI2, WCH),
                         lambda t, pR, sR, gR, cR, *_: (gR[t], cR[t])),
            pl.BlockSpec((1, F), lambda t, *_: (0, 0)),
            pl.BlockSpec((F, F), lambda t, *_: (0, 0)),
            pl.BlockSpec((1, F), lambda t, *_: (0, 0)),
            pl.BlockSpec((F, C), lambda t, *_: (0, 0)),
            pl.BlockSpec((1, C), lambda t, *_: (0, 0)),
        ],
        out_specs=pl.BlockSpec((BI2, C),
                               lambda t, pR, sR, gR, *_: (gR[t], 0)),
        scratch_shapes=[
            pltpu.VMEM((N, 32), jnp.float32),
            pltpu.VMEM((BI2, C), jnp.float32),
        ],
    )
    out = pl.pallas_call(
        _fused_kernel,
        grid_spec=grid_spec,
        out_shape=jax.ShapeDtypeStruct((N, C), jnp.float32),
    )(_P_ARR, _S_ARR, _G_ARR, _C_ARR, _F_ARR, _L_ARR,
      adj, adj, z_eff, W1, b1r, W2, b2r)
    return out
